# baseline (speedup 1.0000x reference)
import jax
import jax.numpy as jnp
from jax.experimental import pallas as pl
from jax.experimental.pallas import tpu as pltpu

VARIANCE_FOCUS = 0.85

_ROWS = 8192
_COLS = 512
_BLK_ROWS = 512
_GRID = _ROWS // _BLK_ROWS


def _silog_body(est_ref, gt_ref, out_ref, acc_ref):
    i = pl.program_id(0)

    @pl.when(i == 0)
    def _init():
        acc_ref[0] = 0.0
        acc_ref[1] = 0.0
        acc_ref[2] = 0.0

    est = est_ref[...]
    gt = gt_ref[...]
    mask = gt > 1.0
    d = jnp.where(
        mask,
        jnp.log(jnp.where(mask, est, 1.0)) - jnp.log(jnp.where(mask, gt, 1.0)),
        0.0,
    )
    acc_ref[0] += jnp.sum(d)
    acc_ref[1] += jnp.sum(d * d)
    acc_ref[2] += jnp.sum(mask.astype(jnp.float32))

    @pl.when(i == _GRID - 1)
    def _fin():
        n = acc_ref[2]
        mean_d = acc_ref[0] / n
        mean_d2 = acc_ref[1] / n
        out_ref[0] = jnp.sqrt(mean_d2 - VARIANCE_FOCUS * mean_d * mean_d) * 10.0


def kernel(depth_est, depth_gt):
    est2d = depth_est.reshape(_ROWS, _COLS)
    gt2d = depth_gt.reshape(_ROWS, _COLS)
    out = pl.pallas_call(
        _silog_body,
        grid=(_GRID,),
        in_specs=[
            pl.BlockSpec((_BLK_ROWS, _COLS), lambda i: (i, 0)),
            pl.BlockSpec((_BLK_ROWS, _COLS), lambda i: (i, 0)),
        ],
        out_specs=pl.BlockSpec(memory_space=pltpu.SMEM),
        out_shape=jax.ShapeDtypeStruct((1,), jnp.float32),
        scratch_shapes=[pltpu.SMEM((3,), jnp.float32)],
    )(est2d, gt2d)
    return out[0]


# speedup vs baseline: 1.3815x; 1.3815x over previous
import jax
import jax.numpy as jnp
from jax.experimental import pallas as pl
from jax.experimental.pallas import tpu as pltpu

VARIANCE_FOCUS = 0.85

_ROWS = 8192
_COLS = 512
_BLK_ROWS = 2048
_GRID = _ROWS // _BLK_ROWS


def _silog_body(est_ref, gt_ref, out_ref, acc_ref):
    i = pl.program_id(0)

    @pl.when(i == 0)
    def _init():
        acc_ref[0] = 0.0
        acc_ref[1] = 0.0
        acc_ref[2] = 0.0

    est = est_ref[...]
    gt = gt_ref[...]
    mask = gt > 1.0
    d = jnp.log(jnp.where(mask, est / gt, 1.0))
    acc_ref[0] += jnp.sum(d)
    acc_ref[1] += jnp.sum(d * d)
    acc_ref[2] += jnp.sum(mask.astype(jnp.float32))

    @pl.when(i == _GRID - 1)
    def _fin():
        n = acc_ref[2]
        mean_d = acc_ref[0] / n
        mean_d2 = acc_ref[1] / n
        out_ref[0] = jnp.sqrt(mean_d2 - VARIANCE_FOCUS * mean_d * mean_d) * 10.0


def kernel(depth_est, depth_gt):
    est2d = depth_est.reshape(_ROWS, _COLS)
    gt2d = depth_gt.reshape(_ROWS, _COLS)
    out = pl.pallas_call(
        _silog_body,
        grid=(_GRID,),
        in_specs=[
            pl.BlockSpec((_BLK_ROWS, _COLS), lambda i: (i, 0)),
            pl.BlockSpec((_BLK_ROWS, _COLS), lambda i: (i, 0)),
        ],
        out_specs=pl.BlockSpec(memory_space=pltpu.SMEM),
        out_shape=jax.ShapeDtypeStruct((1,), jnp.float32),
        scratch_shapes=[pltpu.SMEM((3,), jnp.float32)],
    )(est2d, gt2d)
    return out[0]
